# initial kernel scaffold (unmeasured)
import jax
import jax.numpy as jnp
from jax import lax
from jax.experimental import pallas as pl
from jax.experimental.pallas import tpu as pltpu

N_DEV = 4


def kernel(partial, resid, gamma):
    m, n = resid.shape
    partial2d = partial.reshape(m, n)
    gamma2d = gamma.reshape(1, n)
    P = m // N_DEV

    def body(partial_ref, resid_ref, gamma_ref, out_ref,
             comm_ref, send_sems, recv_sems):
        my = lax.axis_index("i")
        left = (my + N_DEV - 1) % N_DEV
        right = (my + 1) % N_DEV

        barrier_sem = pltpu.get_barrier_semaphore()
        for nbr in (left, right):
            pl.semaphore_signal(
                barrier_sem, inc=1,
                device_id=(nbr,), device_id_type=pl.DeviceIdType.MESH,
            )
        pl.semaphore_wait(barrier_sem, 2)

        ch0 = (my + 3) % N_DEV
        comm_ref[0] = partial_ref[pl.ds(ch0 * P, P), :]
        for s in range(N_DEV - 1):
            send_slot = s % 2
            recv_slot = (s + 1) % 2
            rdma = pltpu.make_async_remote_copy(
                src_ref=comm_ref.at[send_slot],
                dst_ref=comm_ref.at[recv_slot],
                send_sem=send_sems.at[send_slot],
                recv_sem=recv_sems.at[recv_slot],
                device_id=(right,),
                device_id_type=pl.DeviceIdType.MESH,
            )
            rdma.start()
            rdma.wait()
            ch = (my + 2 - s) % N_DEV
            comm_ref[recv_slot] = (
                comm_ref[recv_slot] + partial_ref[pl.ds(ch * P, P), :]
            )

        red_slot = (N_DEV - 1) % 2
        y = comm_ref[red_slot] + resid_ref[pl.ds(my * P, P), :]
        rms = jnp.sqrt(jnp.mean(y * y, axis=-1, keepdims=True) + 1e-6)
        z = y / rms * gamma_ref[...]
        out_ref[pl.ds(my * P, P), :] = z
        comm_ref[0] = z

        for h in range(N_DEV - 1):
            send_slot = h % 2
            recv_slot = (h + 1) % 2
            rdma = pltpu.make_async_remote_copy(
                src_ref=comm_ref.at[send_slot],
                dst_ref=comm_ref.at[recv_slot],
                send_sem=send_sems.at[send_slot],
                recv_sem=recv_sems.at[recv_slot],
                device_id=(right,),
                device_id_type=pl.DeviceIdType.MESH,
            )
            rdma.start()
            rdma.wait()
            origin = (my + 3 - h) % N_DEV
            out_ref[pl.ds(origin * P, P), :] = comm_ref[recv_slot]

    return pl.pallas_call(
        body,
        out_shape=jax.ShapeDtypeStruct((m, n), jnp.float32),
        in_specs=[
            pl.BlockSpec(memory_space=pltpu.VMEM),
            pl.BlockSpec(memory_space=pltpu.VMEM),
            pl.BlockSpec(memory_space=pltpu.VMEM),
        ],
        out_specs=pl.BlockSpec(memory_space=pltpu.VMEM),
        scratch_shapes=[
            pltpu.VMEM((2, P, n), jnp.float32),
            pltpu.SemaphoreType.DMA((2,)),
            pltpu.SemaphoreType.DMA((2,)),
        ],
        compiler_params=pltpu.CompilerParams(collective_id=0),
    )(partial2d, resid, gamma2d)


# baseline (device time: 295602 ns/iter reference)
import jax
import jax.numpy as jnp
from jax import lax
from jax.experimental import pallas as pl
from jax.experimental.pallas import tpu as pltpu

N_DEV = 4


def kernel(partial, resid, gamma):
    m, n = resid.shape
    partial2d = partial.reshape(m, n)
    gamma2d = gamma.reshape(1, n)
    P = m // N_DEV

    def body(partial_ref, resid_ref, gamma_ref, out_ref,
             comm_ref, pchunk_ref, rchunk_ref,
             send_sems, recv_sems, load_sem, rload_sem, store_sem):
        my = lax.axis_index("i")
        left = (my + N_DEV - 1) % N_DEV
        right = (my + 1) % N_DEV

        barrier_sem = pltpu.get_barrier_semaphore()
        for nbr in (left, right):
            pl.semaphore_signal(
                barrier_sem, inc=1,
                device_id=(nbr,), device_id_type=pl.DeviceIdType.MESH,
            )
        pl.semaphore_wait(barrier_sem, 2)

        rload = pltpu.make_async_copy(
            resid_ref.at[pl.ds(my * P, P), :], rchunk_ref, rload_sem,
        )
        rload.start()

        ch0 = (my + 3) % N_DEV
        load0 = pltpu.make_async_copy(
            partial_ref.at[pl.ds(ch0 * P, P), :], comm_ref.at[0], load_sem,
        )
        load0.start()
        load0.wait()
        for s in range(N_DEV - 1):
            send_slot = s % 2
            recv_slot = (s + 1) % 2
            rdma = pltpu.make_async_remote_copy(
                src_ref=comm_ref.at[send_slot],
                dst_ref=comm_ref.at[recv_slot],
                send_sem=send_sems.at[send_slot],
                recv_sem=recv_sems.at[recv_slot],
                device_id=(right,),
                device_id_type=pl.DeviceIdType.MESH,
            )
            rdma.start()
            ch = (my + 2 - s) % N_DEV
            load = pltpu.make_async_copy(
                partial_ref.at[pl.ds(ch * P, P), :], pchunk_ref, load_sem,
            )
            load.start()
            load.wait()
            rdma.wait()
            comm_ref[recv_slot] = comm_ref[recv_slot] + pchunk_ref[...]

        red_slot = (N_DEV - 1) % 2
        rload.wait()
        y = comm_ref[red_slot] + rchunk_ref[...]
        rms = jnp.sqrt(jnp.mean(y * y, axis=-1, keepdims=True) + 1e-6)
        comm_ref[0] = y / rms * gamma_ref[...]
        store0 = pltpu.make_async_copy(
            comm_ref.at[0], out_ref.at[pl.ds(my * P, P), :], store_sem,
        )
        store0.start()
        store0.wait()

        for h in range(N_DEV - 1):
            send_slot = h % 2
            recv_slot = (h + 1) % 2
            rdma = pltpu.make_async_remote_copy(
                src_ref=comm_ref.at[send_slot],
                dst_ref=comm_ref.at[recv_slot],
                send_sem=send_sems.at[send_slot],
                recv_sem=recv_sems.at[recv_slot],
                device_id=(right,),
                device_id_type=pl.DeviceIdType.MESH,
            )
            rdma.start()
            rdma.wait()
            origin = (my + 3 - h) % N_DEV
            store = pltpu.make_async_copy(
                comm_ref.at[recv_slot],
                out_ref.at[pl.ds(origin * P, P), :],
                store_sem,
            )
            store.start()
            store.wait()

    return pl.pallas_call(
        body,
        out_shape=jax.ShapeDtypeStruct((m, n), jnp.float32),
        in_specs=[
            pl.BlockSpec(memory_space=pl.ANY),
            pl.BlockSpec(memory_space=pl.ANY),
            pl.BlockSpec(memory_space=pltpu.VMEM),
        ],
        out_specs=pl.BlockSpec(memory_space=pl.ANY),
        scratch_shapes=[
            pltpu.VMEM((2, P, n), jnp.float32),
            pltpu.VMEM((P, n), jnp.float32),
            pltpu.VMEM((P, n), jnp.float32),
            pltpu.SemaphoreType.DMA((2,)),
            pltpu.SemaphoreType.DMA((2,)),
            pltpu.SemaphoreType.DMA,
            pltpu.SemaphoreType.DMA,
            pltpu.SemaphoreType.DMA,
        ],
        compiler_params=pltpu.CompilerParams(collective_id=0),
    )(partial2d, resid, gamma2d)


# device time: 161131 ns/iter; 1.8345x vs baseline; 1.8345x over previous
import jax
import jax.numpy as jnp
from jax import lax
from jax.experimental import pallas as pl
from jax.experimental.pallas import tpu as pltpu

N_DEV = 4


def kernel(partial, resid, gamma):
    m, n = resid.shape
    partial2d = partial.reshape(m, n)
    gamma2d = gamma.reshape(1, n)
    P = m // N_DEV
    H = P // 2

    def body(partial_ref, resid_ref, gamma_ref, out_ref,
             cw_ref, ccw_ref, pcw_ref, pccw_ref, rchunk_ref,
             send_cw, recv_cw, send_ccw, recv_ccw,
             load_cw_sem, load_ccw_sem, rload_sem, store_sem):
        my = lax.axis_index("i")
        left = (my + N_DEV - 1) % N_DEV
        right = (my + 1) % N_DEV

        barrier_sem = pltpu.get_barrier_semaphore()
        for nbr in (left, right):
            pl.semaphore_signal(
                barrier_sem, inc=1,
                device_id=(nbr,), device_id_type=pl.DeviceIdType.MESH,
            )
        pl.semaphore_wait(barrier_sem, 2)

        rload = pltpu.make_async_copy(
            resid_ref.at[pl.ds(my * P, P), :], rchunk_ref, rload_sem,
        )
        rload.start()

        ch_cw0 = (my + 3) % N_DEV
        ch_ccw0 = (my + 1) % N_DEV
        load_cw0 = pltpu.make_async_copy(
            partial_ref.at[pl.ds(ch_cw0 * P, H), :], cw_ref.at[0], load_cw_sem,
        )
        load_ccw0 = pltpu.make_async_copy(
            partial_ref.at[pl.ds(ch_ccw0 * P + H, H), :], ccw_ref.at[0],
            load_ccw_sem,
        )
        load_cw0.start()
        load_ccw0.start()
        load_cw0.wait()
        load_ccw0.wait()

        for s in range(N_DEV - 1):
            ss = s % 2
            rs = (s + 1) % 2
            rdma_cw = pltpu.make_async_remote_copy(
                src_ref=cw_ref.at[ss], dst_ref=cw_ref.at[rs],
                send_sem=send_cw.at[ss], recv_sem=recv_cw.at[rs],
                device_id=(right,), device_id_type=pl.DeviceIdType.MESH,
            )
            rdma_ccw = pltpu.make_async_remote_copy(
                src_ref=ccw_ref.at[ss], dst_ref=ccw_ref.at[rs],
                send_sem=send_ccw.at[ss], recv_sem=recv_ccw.at[rs],
                device_id=(left,), device_id_type=pl.DeviceIdType.MESH,
            )
            rdma_cw.start()
            rdma_ccw.start()
            ch_cw = (my + 2 - s) % N_DEV
            ch_ccw = (my + 2 + s) % N_DEV
            load_cw = pltpu.make_async_copy(
                partial_ref.at[pl.ds(ch_cw * P, H), :], pcw_ref, load_cw_sem,
            )
            load_ccw = pltpu.make_async_copy(
                partial_ref.at[pl.ds(ch_ccw * P + H, H), :], pccw_ref,
                load_ccw_sem,
            )
            load_cw.start()
            load_ccw.start()
            load_cw.wait()
            load_ccw.wait()
            rdma_cw.wait()
            rdma_ccw.wait()
            cw_ref[rs] = cw_ref[rs] + pcw_ref[...]
            ccw_ref[rs] = ccw_ref[rs] + pccw_ref[...]

        rload.wait()
        red = (N_DEV - 1) % 2
        y_t = cw_ref[red] + rchunk_ref[0:H, :]
        rms_t = jnp.sqrt(jnp.mean(y_t * y_t, axis=-1, keepdims=True) + 1e-6)
        cw_ref[0] = y_t / rms_t * gamma_ref[...]
        y_b = ccw_ref[red] + rchunk_ref[H:2 * H, :]
        rms_b = jnp.sqrt(jnp.mean(y_b * y_b, axis=-1, keepdims=True) + 1e-6)
        ccw_ref[0] = y_b / rms_b * gamma_ref[...]

        store_t = pltpu.make_async_copy(
            cw_ref.at[0], out_ref.at[pl.ds(my * P, H), :], store_sem,
        )
        store_b = pltpu.make_async_copy(
            ccw_ref.at[0], out_ref.at[pl.ds(my * P + H, H), :], store_sem,
        )
        store_t.start()
        store_b.start()
        store_t.wait()
        store_b.wait()

        for h in range(N_DEV - 1):
            ss = h % 2
            rs = (h + 1) % 2
            rdma_cw = pltpu.make_async_remote_copy(
                src_ref=cw_ref.at[ss], dst_ref=cw_ref.at[rs],
                send_sem=send_cw.at[ss], recv_sem=recv_cw.at[rs],
                device_id=(right,), device_id_type=pl.DeviceIdType.MESH,
            )
            rdma_ccw = pltpu.make_async_remote_copy(
                src_ref=ccw_ref.at[ss], dst_ref=ccw_ref.at[rs],
                send_sem=send_ccw.at[ss], recv_sem=recv_ccw.at[rs],
                device_id=(left,), device_id_type=pl.DeviceIdType.MESH,
            )
            rdma_cw.start()
            rdma_ccw.start()
            rdma_cw.wait()
            rdma_ccw.wait()
            origin_cw = (my + 3 - h) % N_DEV
            origin_ccw = (my + 1 + h) % N_DEV
            store_cw = pltpu.make_async_copy(
                cw_ref.at[rs], out_ref.at[pl.ds(origin_cw * P, H), :],
                store_sem,
            )
            store_ccw = pltpu.make_async_copy(
                ccw_ref.at[rs],
                out_ref.at[pl.ds(origin_ccw * P + H, H), :],
                store_sem,
            )
            store_cw.start()
            store_ccw.start()
            store_cw.wait()
            store_ccw.wait()

    return pl.pallas_call(
        body,
        out_shape=jax.ShapeDtypeStruct((m, n), jnp.float32),
        in_specs=[
            pl.BlockSpec(memory_space=pl.ANY),
            pl.BlockSpec(memory_space=pl.ANY),
            pl.BlockSpec(memory_space=pltpu.VMEM),
        ],
        out_specs=pl.BlockSpec(memory_space=pl.ANY),
        scratch_shapes=[
            pltpu.VMEM((2, H, n), jnp.float32),
            pltpu.VMEM((2, H, n), jnp.float32),
            pltpu.VMEM((H, n), jnp.float32),
            pltpu.VMEM((H, n), jnp.float32),
            pltpu.VMEM((P, n), jnp.float32),
            pltpu.SemaphoreType.DMA((2,)),
            pltpu.SemaphoreType.DMA((2,)),
            pltpu.SemaphoreType.DMA((2,)),
            pltpu.SemaphoreType.DMA((2,)),
            pltpu.SemaphoreType.DMA,
            pltpu.SemaphoreType.DMA,
            pltpu.SemaphoreType.DMA,
            pltpu.SemaphoreType.DMA,
        ],
        compiler_params=pltpu.CompilerParams(collective_id=0),
    )(partial2d, resid, gamma2d)


# device time: 160394 ns/iter; 1.8430x vs baseline; 1.0046x over previous
import jax
import jax.numpy as jnp
from jax import lax
from jax.experimental import pallas as pl
from jax.experimental.pallas import tpu as pltpu

N_DEV = 4


def kernel(partial, resid, gamma):
    m, n = resid.shape
    partial2d = partial.reshape(m, n)
    gamma2d = gamma.reshape(1, n)
    P = m // N_DEV
    H = P // 2

    def body(partial_ref, resid_ref, gamma_ref, out_ref,
             cw_ref, ccw_ref, pcw_ref, pccw_ref, rchunk_ref,
             send_cw, recv_cw, send_ccw, recv_ccw,
             load_cw_sem, load_ccw_sem, rload_sem, store_sem):
        my = lax.axis_index("i")
        left = (my + N_DEV - 1) % N_DEV
        right = (my + 1) % N_DEV

        rload = pltpu.make_async_copy(
            resid_ref.at[pl.ds(my * P, P), :], rchunk_ref, rload_sem,
        )
        rload.start()

        ch_cw0 = (my + 3) % N_DEV
        ch_ccw0 = (my + 1) % N_DEV
        load_cw0 = pltpu.make_async_copy(
            partial_ref.at[pl.ds(ch_cw0 * P, H), :], cw_ref.at[0], load_cw_sem,
        )
        load_ccw0 = pltpu.make_async_copy(
            partial_ref.at[pl.ds(ch_ccw0 * P + H, H), :], ccw_ref.at[0],
            load_ccw_sem,
        )
        load_cw0.start()
        load_ccw0.start()

        barrier_sem = pltpu.get_barrier_semaphore()
        for nbr in (left, right):
            pl.semaphore_signal(
                barrier_sem, inc=1,
                device_id=(nbr,), device_id_type=pl.DeviceIdType.MESH,
            )
        pl.semaphore_wait(barrier_sem, 2)

        load_cw0.wait()
        load_ccw0.wait()

        for s in range(N_DEV - 1):
            ss = s % 2
            rs = (s + 1) % 2
            rdma_cw = pltpu.make_async_remote_copy(
                src_ref=cw_ref.at[ss], dst_ref=cw_ref.at[rs],
                send_sem=send_cw.at[ss], recv_sem=recv_cw.at[rs],
                device_id=(right,), device_id_type=pl.DeviceIdType.MESH,
            )
            rdma_ccw = pltpu.make_async_remote_copy(
                src_ref=ccw_ref.at[ss], dst_ref=ccw_ref.at[rs],
                send_sem=send_ccw.at[ss], recv_sem=recv_ccw.at[rs],
                device_id=(left,), device_id_type=pl.DeviceIdType.MESH,
            )
            rdma_cw.start()
            rdma_ccw.start()
            ch_cw = (my + 2 - s) % N_DEV
            ch_ccw = (my + 2 + s) % N_DEV
            load_cw = pltpu.make_async_copy(
                partial_ref.at[pl.ds(ch_cw * P, H), :], pcw_ref, load_cw_sem,
            )
            load_ccw = pltpu.make_async_copy(
                partial_ref.at[pl.ds(ch_ccw * P + H, H), :], pccw_ref,
                load_ccw_sem,
            )
            load_cw.start()
            load_ccw.start()
            load_cw.wait()
            load_ccw.wait()
            rdma_cw.wait()
            rdma_ccw.wait()
            cw_ref[rs] = cw_ref[rs] + pcw_ref[...]
            ccw_ref[rs] = ccw_ref[rs] + pccw_ref[...]

        rload.wait()
        red = (N_DEV - 1) % 2
        pending_stores = []

        y_t = cw_ref[red] + rchunk_ref[0:H, :]
        rms_t = jnp.sqrt(jnp.mean(y_t * y_t, axis=-1, keepdims=True) + 1e-6)
        cw_ref[0] = y_t / rms_t * gamma_ref[...]
        ag_cw0 = pltpu.make_async_remote_copy(
            src_ref=cw_ref.at[0], dst_ref=cw_ref.at[1],
            send_sem=send_cw.at[0], recv_sem=recv_cw.at[1],
            device_id=(right,), device_id_type=pl.DeviceIdType.MESH,
        )
        ag_cw0.start()

        y_b = ccw_ref[red] + rchunk_ref[H:2 * H, :]
        rms_b = jnp.sqrt(jnp.mean(y_b * y_b, axis=-1, keepdims=True) + 1e-6)
        ccw_ref[0] = y_b / rms_b * gamma_ref[...]
        ag_ccw0 = pltpu.make_async_remote_copy(
            src_ref=ccw_ref.at[0], dst_ref=ccw_ref.at[1],
            send_sem=send_ccw.at[0], recv_sem=recv_ccw.at[1],
            device_id=(left,), device_id_type=pl.DeviceIdType.MESH,
        )
        ag_ccw0.start()

        store_t = pltpu.make_async_copy(
            cw_ref.at[0], out_ref.at[pl.ds(my * P, H), :], store_sem,
        )
        store_b = pltpu.make_async_copy(
            ccw_ref.at[0], out_ref.at[pl.ds(my * P + H, H), :], store_sem,
        )
        store_t.start()
        store_b.start()
        pending_stores += [store_t, store_b]

        ag_cw0.wait()
        ag_ccw0.wait()
        for h in range(N_DEV - 1):
            rs = (h + 1) % 2
            origin_cw = (my + 3 - h) % N_DEV
            origin_ccw = (my + 1 + h) % N_DEV
            store_cw = pltpu.make_async_copy(
                cw_ref.at[rs], out_ref.at[pl.ds(origin_cw * P, H), :],
                store_sem,
            )
            store_ccw = pltpu.make_async_copy(
                ccw_ref.at[rs],
                out_ref.at[pl.ds(origin_ccw * P + H, H), :],
                store_sem,
            )
            store_cw.start()
            store_ccw.start()
            pending_stores += [store_cw, store_ccw]
            if h == N_DEV - 2:
                break
            ss = (h + 1) % 2
            nrs = h % 2
            rdma_cw = pltpu.make_async_remote_copy(
                src_ref=cw_ref.at[ss], dst_ref=cw_ref.at[nrs],
                send_sem=send_cw.at[ss], recv_sem=recv_cw.at[nrs],
                device_id=(right,), device_id_type=pl.DeviceIdType.MESH,
            )
            rdma_ccw = pltpu.make_async_remote_copy(
                src_ref=ccw_ref.at[ss], dst_ref=ccw_ref.at[nrs],
                send_sem=send_ccw.at[ss], recv_sem=recv_ccw.at[nrs],
                device_id=(left,), device_id_type=pl.DeviceIdType.MESH,
            )
            rdma_cw.start()
            rdma_ccw.start()
            rdma_cw.wait()
            rdma_ccw.wait()

        for st in pending_stores:
            st.wait()

    return pl.pallas_call(
        body,
        out_shape=jax.ShapeDtypeStruct((m, n), jnp.float32),
        in_specs=[
            pl.BlockSpec(memory_space=pl.ANY),
            pl.BlockSpec(memory_space=pl.ANY),
            pl.BlockSpec(memory_space=pltpu.VMEM),
        ],
        out_specs=pl.BlockSpec(memory_space=pl.ANY),
        scratch_shapes=[
            pltpu.VMEM((2, H, n), jnp.float32),
            pltpu.VMEM((2, H, n), jnp.float32),
            pltpu.VMEM((H, n), jnp.float32),
            pltpu.VMEM((H, n), jnp.float32),
            pltpu.VMEM((P, n), jnp.float32),
            pltpu.SemaphoreType.DMA((2,)),
            pltpu.SemaphoreType.DMA((2,)),
            pltpu.SemaphoreType.DMA((2,)),
            pltpu.SemaphoreType.DMA((2,)),
            pltpu.SemaphoreType.DMA,
            pltpu.SemaphoreType.DMA,
            pltpu.SemaphoreType.DMA,
            pltpu.SemaphoreType.DMA,
        ],
        compiler_params=pltpu.CompilerParams(collective_id=0),
    )(partial2d, resid, gamma2d)
